# baseline (speedup 1.0000x reference)
import functools

import jax
import jax.numpy as jnp
from jax import lax
from jax.experimental import pallas as pl
from jax.experimental.pallas import tpu as pltpu
from jax.experimental.pallas import tpu_sc as plsc

NN = 10000
D = 128
DH = 64
NE = 320000
NC, NS, L = 2, 16, 16
CH = 64
NCH = 320
W = 16
PER_W = NCH * CH
EPAD = NS * PER_W
NBUF = 8
LA = 4
ZR = 32
RPS = 640
NNP = NS * RPS
NB = RPS // CH


def _lightgcn_sc(x, src3, dst3, val3):
    mesh = plsc.VectorSubcoreMesh(core_axis_name="c", subcore_axis_name="s")

    @functools.partial(
        pl.kernel,
        mesh=mesh,
        compiler_params=pltpu.CompilerParams(use_tc_tiling_on_sc=False),
        out_type=jax.ShapeDtypeStruct((3, NC, NN, DH), jnp.float32),
        scratch_types=[
            pltpu.VMEM((2, W, CH), jnp.int32),
            pltpu.VMEM((2, W, CH), jnp.int32),
            pltpu.VMEM((2, W, CH), jnp.float32),
            pltpu.VMEM((NBUF, CH, DH), jnp.float32),
            pltpu.VMEM((ZR, DH), jnp.float32),
            pltpu.VMEM_SHARED((NNP, DH), jnp.float32),
            pltpu.VMEM_SHARED((NNP, DH), jnp.float32),
        ] + [pltpu.SemaphoreType.DMA] * (2 * NBUF + 1),
    )
    def k(x_hbm, src_hbm, dst_hbm, val_hbm, out_hbm,
          src_v, dst_v, val_v, rows_v, zero_v, a_sh, b_sh,
          *sems):
        gsem = list(sems[:NBUF])
        ssem = list(sems[NBUF:2 * NBUF])
        isem = sems[2 * NBUF]
        c = lax.axis_index("c")
        s = lax.axis_index("s")
        r0 = s * RPS
        xh = x_hbm.at[c]

        z16 = jnp.zeros((L,), jnp.float32)

        @pl.loop(0, ZR)
        def _(e):
            for j in range(DH // L):
                zero_v[e, pl.ds(j * L, L)] = z16

        pltpu.sync_copy(xh.at[pl.ds(r0, 384)], a_sh.at[pl.ds(r0, 384)])

        @pl.when(s < NS - 1)
        def _():
            pltpu.sync_copy(xh.at[pl.ds(r0 + 384, RPS - 384)],
                            a_sh.at[pl.ds(r0 + 384, RPS - 384)])

        @pl.when(s == NS - 1)
        def _():
            pltpu.sync_copy(xh.at[pl.ds(r0 + 384, 16)],
                            a_sh.at[pl.ds(r0 + 384, 16)])

        for t in range(RPS // ZR):
            pltpu.sync_copy(zero_v, b_sh.at[pl.ds(r0 + t * ZR, ZR)])

        plsc.subcore_barrier()

        def idx_load(p, par):
            pltpu.async_copy(src_hbm.at[s, pl.ds(p * W, W)], src_v.at[par],
                             isem)
            pltpu.async_copy(dst_hbm.at[s, pl.ds(p * W, W)], dst_v.at[par],
                             isem)
            pltpu.async_copy(val_hbm.at[s, pl.ds(p * W, W)], val_v.at[par],
                             isem)

        def idx_wait():
            pltpu.make_async_copy(src_hbm.at[s, pl.ds(0, W)], src_v.at[0],
                                  isem).wait()
            pltpu.make_async_copy(dst_hbm.at[s, pl.ds(0, W)], dst_v.at[0],
                                  isem).wait()
            pltpu.make_async_copy(val_hbm.at[s, pl.ds(0, W)], val_v.at[0],
                                  isem).wait()

        def edge_loop(src_ref, dst_ref):
            def issue_gather(par, cidx, b):
                pltpu.async_copy(src_ref.at[src_v.at[par, cidx]],
                                 rows_v.at[b], gsem[b])

            def wait_gather(b):
                pltpu.make_async_copy(src_ref.at[pl.ds(0, CH)], rows_v.at[b],
                                      gsem[b]).wait()

            def issue_scatter(par, cidx, b):
                pltpu.async_copy(rows_v.at[b], dst_ref.at[dst_v.at[par, cidx]],
                                 ssem[b], add=True)

            def wait_scatter(b):
                pltpu.make_async_copy(rows_v.at[b], dst_ref.at[pl.ds(0, CH)],
                                      ssem[b]).wait()

            def scale(par, cidx, b):
                @pl.loop(0, CH // L)
                def _(g):
                    vals16 = val_v[par, cidx, pl.ds(g * L, L)]
                    for l in range(L):
                        bidx = jnp.full((L,), l, jnp.int32)
                        v = vals16.at[bidx].get(mode="promise_in_bounds")
                        e = g * L + l
                        for j in range(DH // L):
                            sl = pl.ds(j * L, L)
                            rows_v[b, e, sl] = rows_v[b, e, sl] * v

            idx_load(0, 0)

            @pl.loop(0, NCH // W)
            def _(p):
                par = lax.rem(p, 2)
                idx_wait()

                @pl.when(p < NCH // W - 1)
                def _():
                    idx_load(p + 1, 1 - par)

                for b in range(LA):
                    issue_gather(par, b, b)

                @pl.loop(0, W // NBUF)
                def _(k_):
                    for b in range(NBUF):
                        cidx = k_ * NBUF + b
                        bn = (b + LA) % NBUF
                        if b >= LA:
                            wait_scatter(bn)
                        else:
                            @pl.when(k_ > 0)
                            def _():
                                wait_scatter(bn)
                        if b < NBUF - LA:
                            issue_gather(par, cidx + LA, bn)
                        else:
                            @pl.when(k_ < W // NBUF - 1)
                            def _():
                                issue_gather(par, cidx + LA, bn)
                        wait_gather(b)
                        scale(par, cidx, b)
                        issue_scatter(par, cidx, b)

                for b in range(LA, NBUF):
                    wait_scatter(b)

        def write_layer(dref, li):
            pltpu.sync_copy(dref.at[pl.ds(r0, 384)],
                            out_hbm.at[li, c, pl.ds(r0, 384)])

            @pl.when(s < NS - 1)
            def _():
                pltpu.sync_copy(dref.at[pl.ds(r0 + 384, RPS - 384)],
                                out_hbm.at[li, c, pl.ds(r0 + 384, RPS - 384)])

            @pl.when(s == NS - 1)
            def _():
                pltpu.sync_copy(dref.at[pl.ds(r0 + 384, 16)],
                                out_hbm.at[li, c, pl.ds(r0 + 384, 16)])

        def zero_slice(dref):
            for t in range(RPS // ZR):
                pltpu.sync_copy(zero_v, dref.at[pl.ds(r0 + t * ZR, ZR)])

        edge_loop(a_sh, b_sh)
        plsc.subcore_barrier()
        write_layer(b_sh, 0)
        zero_slice(a_sh)
        plsc.subcore_barrier()
        edge_loop(b_sh, a_sh)
        plsc.subcore_barrier()
        write_layer(a_sh, 1)
        zero_slice(b_sh)
        plsc.subcore_barrier()
        edge_loop(a_sh, b_sh)
        plsc.subcore_barrier()
        write_layer(b_sh, 2)

    return k(x, src3, dst3, val3)


def _sum4_tc(x, layers):
    RB = 2000
    x2 = x.reshape(NC * NN, DH)
    l2 = layers.reshape(3, NC * NN, DH)

    def body(x_ref, l_ref, o_ref):
        o_ref[...] = (x_ref[...] + l_ref[0] + l_ref[1] + l_ref[2])

    out = pl.pallas_call(
        body,
        grid=(NC * NN // RB,),
        in_specs=[pl.BlockSpec((RB, DH), lambda i: (i, 0)),
                  pl.BlockSpec((3, RB, DH), lambda i: (0, i, 0))],
        out_specs=pl.BlockSpec((RB, DH), lambda i: (i, 0)),
        out_shape=jax.ShapeDtypeStruct((NC * NN, DH), jnp.float32),
    )(x2, l2)
    return out.reshape(NC, NN, DH)


def kernel(ebds, adj_edge_index, adj_values):
    pad = EPAD - NE
    src = jnp.concatenate([adj_edge_index[0],
                           jnp.zeros((pad,), jnp.int32)]).reshape(NS, NCH, CH)
    dst = jnp.concatenate([adj_edge_index[1],
                           jnp.zeros((pad,), jnp.int32)]).reshape(NS, NCH, CH)
    val = jnp.concatenate([adj_values,
                           jnp.zeros((pad,), jnp.float32)]).reshape(NS, NCH, CH)

    x = ebds.reshape(NN, NC, DH).transpose(1, 0, 2)
    layers = _lightgcn_sc(x, src, dst, val)
    total = _sum4_tc(x, layers)
    return total.transpose(1, 0, 2).reshape(NN, D)


# speedup vs baseline: 1.0828x; 1.0828x over previous
import functools

import jax
import jax.numpy as jnp
from jax import lax
from jax.experimental import pallas as pl
from jax.experimental.pallas import tpu as pltpu
from jax.experimental.pallas import tpu_sc as plsc

NN = 10000
D = 128
DH = 64
NE = 320000
NC, NS, L = 2, 16, 16
CH = 64
NCH = 320
W = 20
PER_W = NCH * CH
EPAD = NS * PER_W
NBUF = 10
LA = 5
ZR = 16
RPS = 624
TAIL = NN - NS * RPS


def _lightgcn_sc(x, src3, dst3, val3):
    mesh = plsc.VectorSubcoreMesh(core_axis_name="c", subcore_axis_name="s")

    @functools.partial(
        pl.kernel,
        mesh=mesh,
        compiler_params=pltpu.CompilerParams(use_tc_tiling_on_sc=False),
        out_type=jax.ShapeDtypeStruct((3, NC, NN, DH), jnp.float32),
        scratch_types=[
            pltpu.VMEM((2, W, CH), jnp.int32),
            pltpu.VMEM((2, W, CH), jnp.int32),
            pltpu.VMEM((2, W, CH), jnp.float32),
            pltpu.VMEM((NBUF, CH, DH), jnp.float32),
            pltpu.VMEM((ZR, DH), jnp.float32),
            pltpu.VMEM_SHARED((NN, DH), jnp.float32),
            pltpu.VMEM_SHARED((NN, DH), jnp.float32),
        ] + [pltpu.SemaphoreType.DMA] * (2 * NBUF + 1),
    )
    def k(x_hbm, src_hbm, dst_hbm, val_hbm, out_hbm,
          src_v, dst_v, val_v, rows_v, zero_v, a_sh, b_sh,
          *sems):
        gsem = list(sems[:NBUF])
        ssem = list(sems[NBUF:2 * NBUF])
        isem = sems[2 * NBUF]
        c = lax.axis_index("c")
        s = lax.axis_index("s")
        r0 = s * RPS
        xh = x_hbm.at[c]

        z16 = jnp.zeros((L,), jnp.float32)

        @pl.loop(0, ZR)
        def _(e):
            for j in range(DH // L):
                zero_v[e, pl.ds(j * L, L)] = z16

        pltpu.sync_copy(xh.at[pl.ds(r0, RPS)], a_sh.at[pl.ds(r0, RPS)])

        @pl.when(s == NS - 1)
        def _():
            pltpu.sync_copy(xh.at[pl.ds(NS * RPS, TAIL)],
                            a_sh.at[pl.ds(NS * RPS, TAIL)])

        for t in range(RPS // ZR):
            pltpu.sync_copy(zero_v, b_sh.at[pl.ds(r0 + t * ZR, ZR)])

        @pl.when(s == NS - 1)
        def _():
            pltpu.sync_copy(zero_v.at[pl.ds(0, TAIL)],
                            b_sh.at[pl.ds(NS * RPS, TAIL)])

        plsc.subcore_barrier()

        def idx_load(p, par):
            pltpu.async_copy(src_hbm.at[s, pl.ds(p * W, W)], src_v.at[par],
                             isem)
            pltpu.async_copy(dst_hbm.at[s, pl.ds(p * W, W)], dst_v.at[par],
                             isem)
            pltpu.async_copy(val_hbm.at[s, pl.ds(p * W, W)], val_v.at[par],
                             isem)

        def idx_wait():
            pltpu.make_async_copy(src_hbm.at[s, pl.ds(0, W)], src_v.at[0],
                                  isem).wait()
            pltpu.make_async_copy(dst_hbm.at[s, pl.ds(0, W)], dst_v.at[0],
                                  isem).wait()
            pltpu.make_async_copy(val_hbm.at[s, pl.ds(0, W)], val_v.at[0],
                                  isem).wait()

        def edge_loop(src_ref, dst_ref):
            def issue_gather(par, cidx, b):
                pltpu.async_copy(src_ref.at[src_v.at[par, cidx]],
                                 rows_v.at[b], gsem[b])

            def wait_gather(b):
                pltpu.make_async_copy(src_ref.at[pl.ds(0, CH)], rows_v.at[b],
                                      gsem[b]).wait()

            def issue_scatter(par, cidx, b):
                pltpu.async_copy(rows_v.at[b], dst_ref.at[dst_v.at[par, cidx]],
                                 ssem[b], add=True)

            def wait_scatter(b):
                pltpu.make_async_copy(rows_v.at[b], dst_ref.at[pl.ds(0, CH)],
                                      ssem[b]).wait()

            def scale(par, cidx, b):
                @pl.loop(0, CH // L)
                def _(g):
                    vals16 = val_v[par, cidx, pl.ds(g * L, L)]
                    for l in range(L):
                        bidx = jnp.full((L,), l, jnp.int32)
                        v = vals16.at[bidx].get(mode="promise_in_bounds")
                        e = g * L + l
                        for j in range(DH // L):
                            sl = pl.ds(j * L, L)
                            rows_v[b, e, sl] = rows_v[b, e, sl] * v

            idx_load(0, 0)

            @pl.loop(0, NCH // W)
            def _(p):
                par = lax.rem(p, 2)
                idx_wait()

                @pl.when(p < NCH // W - 1)
                def _():
                    idx_load(p + 1, 1 - par)

                for b in range(LA):
                    issue_gather(par, b, b)

                @pl.loop(0, W // NBUF)
                def _(k_):
                    for b in range(NBUF):
                        cidx = k_ * NBUF + b
                        bn = (b + LA) % NBUF
                        if b >= LA:
                            wait_scatter(bn)
                        else:
                            @pl.when(k_ > 0)
                            def _():
                                wait_scatter(bn)
                        if b < NBUF - LA:
                            issue_gather(par, cidx + LA, bn)
                        else:
                            @pl.when(k_ < W // NBUF - 1)
                            def _():
                                issue_gather(par, cidx + LA, bn)
                        wait_gather(b)
                        scale(par, cidx, b)
                        issue_scatter(par, cidx, b)

                for b in range(LA, NBUF):
                    wait_scatter(b)

        def write_layer(dref, li):
            pltpu.sync_copy(dref.at[pl.ds(r0, RPS)],
                            out_hbm.at[li, c, pl.ds(r0, RPS)])

            @pl.when(s == NS - 1)
            def _():
                pltpu.sync_copy(dref.at[pl.ds(NS * RPS, TAIL)],
                                out_hbm.at[li, c, pl.ds(NS * RPS, TAIL)])

        def zero_slice(dref):
            for t in range(RPS // ZR):
                pltpu.sync_copy(zero_v, dref.at[pl.ds(r0 + t * ZR, ZR)])

            @pl.when(s == NS - 1)
            def _():
                pltpu.sync_copy(zero_v.at[pl.ds(0, TAIL)],
                                dref.at[pl.ds(NS * RPS, TAIL)])

        edge_loop(a_sh, b_sh)
        plsc.subcore_barrier()
        write_layer(b_sh, 0)
        zero_slice(a_sh)
        plsc.subcore_barrier()
        edge_loop(b_sh, a_sh)
        plsc.subcore_barrier()
        write_layer(a_sh, 1)
        zero_slice(b_sh)
        plsc.subcore_barrier()
        edge_loop(a_sh, b_sh)
        plsc.subcore_barrier()
        write_layer(b_sh, 2)

    return k(x, src3, dst3, val3)


def _sum4_tc(x, layers):
    RB = 2000
    x2 = x.reshape(NC * NN, DH)
    l2 = layers.reshape(3, NC * NN, DH)

    def body(x_ref, l_ref, o_ref):
        o_ref[...] = (x_ref[...] + l_ref[0] + l_ref[1] + l_ref[2])

    out = pl.pallas_call(
        body,
        grid=(NC * NN // RB,),
        in_specs=[pl.BlockSpec((RB, DH), lambda i: (i, 0)),
                  pl.BlockSpec((3, RB, DH), lambda i: (0, i, 0))],
        out_specs=pl.BlockSpec((RB, DH), lambda i: (i, 0)),
        out_shape=jax.ShapeDtypeStruct((NC * NN, DH), jnp.float32),
    )(x2, l2)
    return out.reshape(NC, NN, DH)


def kernel(ebds, adj_edge_index, adj_values):
    pad = EPAD - NE
    src = jnp.concatenate([adj_edge_index[0],
                           jnp.zeros((pad,), jnp.int32)]).reshape(NS, NCH, CH)
    dst = jnp.concatenate([adj_edge_index[1],
                           jnp.zeros((pad,), jnp.int32)]).reshape(NS, NCH, CH)
    val = jnp.concatenate([adj_values,
                           jnp.zeros((pad,), jnp.float32)]).reshape(NS, NCH, CH)

    x = ebds.reshape(NN, NC, DH).transpose(1, 0, 2)
    layers = _lightgcn_sc(x, src, dst, val)
    total = _sum4_tc(x, layers)
    return total.transpose(1, 0, 2).reshape(NN, D)


# speedup vs baseline: 1.0833x; 1.0004x over previous
import functools

import jax
import jax.numpy as jnp
from jax import lax
from jax.experimental import pallas as pl
from jax.experimental.pallas import tpu as pltpu
from jax.experimental.pallas import tpu_sc as plsc

NN = 10000
D = 128
DH = 64
NE = 320000
NC, NS, L = 2, 16, 16
CH = 64
NCH = 320
W = 20
PER_W = NCH * CH
EPAD = NS * PER_W
NBUF = 10
LA = 5
ZR = 16
RPS = 624
TAIL = NN - NS * RPS


def _lightgcn_sc(x, src3, dst3, val3):
    mesh = plsc.VectorSubcoreMesh(core_axis_name="c", subcore_axis_name="s")

    @functools.partial(
        pl.kernel,
        mesh=mesh,
        compiler_params=pltpu.CompilerParams(use_tc_tiling_on_sc=False),
        out_type=jax.ShapeDtypeStruct((3, NC, NN, DH), jnp.float32),
        scratch_types=[
            pltpu.VMEM((2, W, CH), jnp.int32),
            pltpu.VMEM((2, W, CH), jnp.int32),
            pltpu.VMEM((2, W, CH), jnp.float32),
            pltpu.VMEM((NBUF, CH, DH), jnp.float32),
            pltpu.VMEM((ZR, DH), jnp.float32),
            pltpu.VMEM_SHARED((NN, DH), jnp.float32),
            pltpu.VMEM_SHARED((NN, DH), jnp.float32),
        ] + [pltpu.SemaphoreType.DMA] * (2 * NBUF + 1),
    )
    def k(x_hbm, src_hbm, dst_hbm, val_hbm, out_hbm,
          src_v, dst_v, val_v, rows_v, zero_v, a_sh, b_sh,
          *sems):
        gsem = list(sems[:NBUF])
        ssem = list(sems[NBUF:2 * NBUF])
        isem = sems[2 * NBUF]
        c = lax.axis_index("c")
        s = lax.axis_index("s")
        r0 = s * RPS
        xh = x_hbm.at[c]

        z16 = jnp.zeros((L,), jnp.float32)

        @pl.loop(0, ZR)
        def _(e):
            for j in range(DH // L):
                zero_v[e, pl.ds(j * L, L)] = z16

        pltpu.sync_copy(xh.at[pl.ds(r0, RPS)], a_sh.at[pl.ds(r0, RPS)])

        @pl.when(s == NS - 1)
        def _():
            pltpu.sync_copy(xh.at[pl.ds(NS * RPS, TAIL)],
                            a_sh.at[pl.ds(NS * RPS, TAIL)])

        for t in range(RPS // ZR):
            pltpu.sync_copy(zero_v, b_sh.at[pl.ds(r0 + t * ZR, ZR)])

        @pl.when(s == NS - 1)
        def _():
            pltpu.sync_copy(zero_v.at[pl.ds(0, TAIL)],
                            b_sh.at[pl.ds(NS * RPS, TAIL)])

        plsc.subcore_barrier()

        def idx_load(p, par):
            pltpu.async_copy(src_hbm.at[s, pl.ds(p * W, W)], src_v.at[par],
                             isem)
            pltpu.async_copy(dst_hbm.at[s, pl.ds(p * W, W)], dst_v.at[par],
                             isem)
            pltpu.async_copy(val_hbm.at[s, pl.ds(p * W, W)], val_v.at[par],
                             isem)

        def idx_wait():
            pltpu.make_async_copy(src_hbm.at[s, pl.ds(0, W)], src_v.at[0],
                                  isem).wait()
            pltpu.make_async_copy(dst_hbm.at[s, pl.ds(0, W)], dst_v.at[0],
                                  isem).wait()
            pltpu.make_async_copy(val_hbm.at[s, pl.ds(0, W)], val_v.at[0],
                                  isem).wait()

        def edge_loop(src_ref, dst_ref):
            def issue_gather(par, cidx, b):
                pltpu.async_copy(src_ref.at[src_v.at[par, cidx]],
                                 rows_v.at[b], gsem[b])

            def wait_gather(b):
                pltpu.make_async_copy(src_ref.at[pl.ds(0, CH)], rows_v.at[b],
                                      gsem[b]).wait()

            def issue_scatter(par, cidx, b):
                pltpu.async_copy(rows_v.at[b], dst_ref.at[dst_v.at[par, cidx]],
                                 ssem[b], add=True)

            def wait_scatter(b):
                pltpu.make_async_copy(rows_v.at[b], dst_ref.at[pl.ds(0, CH)],
                                      ssem[b]).wait()

            def scale(par, cidx, b):
                @pl.loop(0, CH // L)
                def _(g):
                    vals16 = val_v[par, cidx, pl.ds(g * L, L)]
                    for l in range(L):
                        bidx = jnp.full((L,), l, jnp.int32)
                        v = vals16.at[bidx].get(mode="promise_in_bounds")
                        e = g * L + l
                        for j in range(DH // L):
                            sl = pl.ds(j * L, L)
                            rows_v[b, e, sl] = rows_v[b, e, sl] * v

            @pl.loop(0, NCH // W)
            def _(p):
                par = 0
                idx_load(p, par)
                idx_wait()

                for b in range(LA):
                    issue_gather(par, b, b)

                @pl.loop(0, W // NBUF)
                def _(k_):
                    for b in range(NBUF):
                        cidx = k_ * NBUF + b
                        bn = (b + LA) % NBUF
                        if b >= LA:
                            wait_scatter(bn)
                        else:
                            @pl.when(k_ > 0)
                            def _():
                                wait_scatter(bn)
                        if b < NBUF - LA:
                            issue_gather(par, cidx + LA, bn)
                        else:
                            @pl.when(k_ < W // NBUF - 1)
                            def _():
                                issue_gather(par, cidx + LA, bn)
                        wait_gather(b)
                        scale(par, cidx, b)
                        issue_scatter(par, cidx, b)

                for b in range(LA, NBUF):
                    wait_scatter(b)

        def write_layer(dref, li):
            pltpu.sync_copy(dref.at[pl.ds(r0, RPS)],
                            out_hbm.at[li, c, pl.ds(r0, RPS)])

            @pl.when(s == NS - 1)
            def _():
                pltpu.sync_copy(dref.at[pl.ds(NS * RPS, TAIL)],
                                out_hbm.at[li, c, pl.ds(NS * RPS, TAIL)])

        def zero_slice(dref):
            for t in range(RPS // ZR):
                pltpu.sync_copy(zero_v, dref.at[pl.ds(r0 + t * ZR, ZR)])

            @pl.when(s == NS - 1)
            def _():
                pltpu.sync_copy(zero_v.at[pl.ds(0, TAIL)],
                                dref.at[pl.ds(NS * RPS, TAIL)])

        edge_loop(a_sh, b_sh)
        plsc.subcore_barrier()
        write_layer(b_sh, 0)
        zero_slice(a_sh)
        plsc.subcore_barrier()
        edge_loop(b_sh, a_sh)
        plsc.subcore_barrier()
        write_layer(a_sh, 1)
        zero_slice(b_sh)
        plsc.subcore_barrier()
        edge_loop(a_sh, b_sh)
        plsc.subcore_barrier()
        write_layer(b_sh, 2)

    return k(x, src3, dst3, val3)


def _sum4_tc(x, layers):
    RB = 2000
    x2 = x.reshape(NC * NN, DH)
    l2 = layers.reshape(3, NC * NN, DH)

    def body(x_ref, l_ref, o_ref):
        o_ref[...] = (x_ref[...] + l_ref[0] + l_ref[1] + l_ref[2])

    out = pl.pallas_call(
        body,
        grid=(NC * NN // RB,),
        in_specs=[pl.BlockSpec((RB, DH), lambda i: (i, 0)),
                  pl.BlockSpec((3, RB, DH), lambda i: (0, i, 0))],
        out_specs=pl.BlockSpec((RB, DH), lambda i: (i, 0)),
        out_shape=jax.ShapeDtypeStruct((NC * NN, DH), jnp.float32),
    )(x2, l2)
    return out.reshape(NC, NN, DH)


def kernel(ebds, adj_edge_index, adj_values):
    pad = EPAD - NE
    src = jnp.concatenate([adj_edge_index[0],
                           jnp.zeros((pad,), jnp.int32)]).reshape(NS, NCH, CH)
    dst = jnp.concatenate([adj_edge_index[1],
                           jnp.zeros((pad,), jnp.int32)]).reshape(NS, NCH, CH)
    val = jnp.concatenate([adj_values,
                           jnp.zeros((pad,), jnp.float32)]).reshape(NS, NCH, CH)

    x = ebds.reshape(NN, NC, DH).transpose(1, 0, 2)
    layers = _lightgcn_sc(x, src, dst, val)
    total = _sum4_tc(x, layers)
    return total.transpose(1, 0, 2).reshape(NN, D)


# speedup vs baseline: 1.0939x; 1.0098x over previous
import functools

import jax
import jax.numpy as jnp
from jax import lax
from jax.experimental import pallas as pl
from jax.experimental.pallas import tpu as pltpu
from jax.experimental.pallas import tpu_sc as plsc

NN = 10000
D = 128
DH = 64
NE = 320000
NC, NS, L = 2, 16, 16
CH = 64
NCH = 320
W = 20
PER_W = NCH * CH
EPAD = NS * PER_W
NBUF = 10
LA = 5
ZR = 32
RPS = 640
NNP = NS * RPS
NB = RPS // CH


def _lightgcn_sc(x, src3, dst3, val3):
    mesh = plsc.VectorSubcoreMesh(core_axis_name="c", subcore_axis_name="s")

    @functools.partial(
        pl.kernel,
        mesh=mesh,
        compiler_params=pltpu.CompilerParams(use_tc_tiling_on_sc=False),
        out_type=jax.ShapeDtypeStruct((3, NC, NN, DH), jnp.float32),
        scratch_types=[
            pltpu.VMEM((W, CH), jnp.int32),
            pltpu.VMEM((W, CH), jnp.int32),
            pltpu.VMEM((W, CH), jnp.float32),
            pltpu.VMEM((NBUF, CH, DH), jnp.float32),
            pltpu.VMEM((ZR, DH), jnp.float32),
            pltpu.VMEM_SHARED((NNP, DH), jnp.float32),
            pltpu.VMEM_SHARED((NNP, DH), jnp.float32),
        ] + [pltpu.SemaphoreType.DMA] * (2 * NBUF + 1),
    )
    def k(x_hbm, src_hbm, dst_hbm, val_hbm, out_hbm,
          src_v, dst_v, val_v, rows_v, zero_v, a_sh, b_sh,
          *sems):
        gsem = list(sems[:NBUF])
        ssem = list(sems[NBUF:2 * NBUF])
        isem = sems[2 * NBUF]
        c = lax.axis_index("c")
        s = lax.axis_index("s")
        r0 = s * RPS
        xh = x_hbm.at[c]

        z16 = jnp.zeros((L,), jnp.float32)

        @pl.loop(0, ZR)
        def _(e):
            for j in range(DH // L):
                zero_v[e, pl.ds(j * L, L)] = z16

        pltpu.sync_copy(xh.at[pl.ds(r0, 384)], a_sh.at[pl.ds(r0, 384)])

        @pl.when(s < NS - 1)
        def _():
            pltpu.sync_copy(xh.at[pl.ds(r0 + 384, RPS - 384)],
                            a_sh.at[pl.ds(r0 + 384, RPS - 384)])

        @pl.when(s == NS - 1)
        def _():
            pltpu.sync_copy(xh.at[pl.ds(r0 + 384, 16)],
                            a_sh.at[pl.ds(r0 + 384, 16)])

        for t in range(RPS // ZR):
            pltpu.sync_copy(zero_v, b_sh.at[pl.ds(r0 + t * ZR, ZR)])

        plsc.subcore_barrier()

        def edge_loop(src_ref, dst_ref):
            def issue_gather(cidx, b):
                pltpu.async_copy(src_ref.at[src_v.at[cidx]], rows_v.at[b],
                                 gsem[b])

            def wait_gather(b):
                pltpu.make_async_copy(src_ref.at[pl.ds(0, CH)], rows_v.at[b],
                                      gsem[b]).wait()

            def issue_scatter(cidx, b):
                pltpu.async_copy(rows_v.at[b], dst_ref.at[dst_v.at[cidx]],
                                 ssem[b], add=True)

            def wait_scatter(b):
                pltpu.make_async_copy(rows_v.at[b], dst_ref.at[pl.ds(0, CH)],
                                      ssem[b]).wait()

            def scale(cidx, b):
                @pl.loop(0, CH // L)
                def _(g):
                    vals16 = val_v[cidx, pl.ds(g * L, L)]
                    for l in range(L):
                        bidx = jnp.full((L,), l, jnp.int32)
                        v = vals16.at[bidx].get(mode="promise_in_bounds")
                        e = g * L + l
                        for j in range(DH // L):
                            sl = pl.ds(j * L, L)
                            rows_v[b, e, sl] = rows_v[b, e, sl] * v

            @pl.loop(0, NCH // W)
            def _(p):
                pltpu.async_copy(src_hbm.at[s, pl.ds(p * W, W)], src_v, isem)
                pltpu.async_copy(dst_hbm.at[s, pl.ds(p * W, W)], dst_v, isem)
                pltpu.async_copy(val_hbm.at[s, pl.ds(p * W, W)], val_v, isem)
                pltpu.make_async_copy(src_hbm.at[s, pl.ds(0, W)], src_v,
                                      isem).wait()
                pltpu.make_async_copy(dst_hbm.at[s, pl.ds(0, W)], dst_v,
                                      isem).wait()
                pltpu.make_async_copy(val_hbm.at[s, pl.ds(0, W)], val_v,
                                      isem).wait()

                for b in range(LA):
                    issue_gather(b, b)

                @pl.loop(0, W // NBUF)
                def _(k_):
                    for b in range(NBUF):
                        cidx = k_ * NBUF + b
                        bn = (b + LA) % NBUF
                        if b >= LA:
                            wait_scatter(bn)
                        else:
                            @pl.when(k_ > 0)
                            def _():
                                wait_scatter(bn)
                        if b < NBUF - LA:
                            issue_gather(cidx + LA, bn)
                        else:
                            @pl.when(k_ < W // NBUF - 1)
                            def _():
                                issue_gather(cidx + LA, bn)
                        wait_gather(b)
                        scale(cidx, b)
                        issue_scatter(cidx, b)

                for b in range(LA, NBUF):
                    wait_scatter(b)

        def write_layer(dref, li):
            pltpu.sync_copy(dref.at[pl.ds(r0, 384)],
                            out_hbm.at[li, c, pl.ds(r0, 384)])

            @pl.when(s < NS - 1)
            def _():
                pltpu.sync_copy(dref.at[pl.ds(r0 + 384, RPS - 384)],
                                out_hbm.at[li, c, pl.ds(r0 + 384, RPS - 384)])

            @pl.when(s == NS - 1)
            def _():
                pltpu.sync_copy(dref.at[pl.ds(r0 + 384, 16)],
                                out_hbm.at[li, c, pl.ds(r0 + 384, 16)])

        def zero_slice(dref):
            for t in range(RPS // ZR):
                pltpu.sync_copy(zero_v, dref.at[pl.ds(r0 + t * ZR, ZR)])

        edge_loop(a_sh, b_sh)
        plsc.subcore_barrier()
        write_layer(b_sh, 0)
        zero_slice(a_sh)
        plsc.subcore_barrier()
        edge_loop(b_sh, a_sh)
        plsc.subcore_barrier()
        write_layer(a_sh, 1)
        zero_slice(b_sh)
        plsc.subcore_barrier()
        edge_loop(a_sh, b_sh)
        plsc.subcore_barrier()
        write_layer(b_sh, 2)

    return k(x, src3, dst3, val3)


def _sum4_tc(x, layers):
    RB = 2000
    x2 = x.reshape(NC * NN, DH)
    l2 = layers.reshape(3, NC * NN, DH)

    def body(x_ref, l_ref, o_ref):
        o_ref[...] = (x_ref[...] + l_ref[0] + l_ref[1] + l_ref[2])

    out = pl.pallas_call(
        body,
        grid=(NC * NN // RB,),
        in_specs=[pl.BlockSpec((RB, DH), lambda i: (i, 0)),
                  pl.BlockSpec((3, RB, DH), lambda i: (0, i, 0))],
        out_specs=pl.BlockSpec((RB, DH), lambda i: (i, 0)),
        out_shape=jax.ShapeDtypeStruct((NC * NN, DH), jnp.float32),
    )(x2, l2)
    return out.reshape(NC, NN, DH)


def kernel(ebds, adj_edge_index, adj_values):
    pad = EPAD - NE
    src = jnp.concatenate([adj_edge_index[0],
                           jnp.zeros((pad,), jnp.int32)]).reshape(NS, NCH, CH)
    dst = jnp.concatenate([adj_edge_index[1],
                           jnp.zeros((pad,), jnp.int32)]).reshape(NS, NCH, CH)
    val = jnp.concatenate([adj_values,
                           jnp.zeros((pad,), jnp.float32)]).reshape(NS, NCH, CH)

    x = ebds.reshape(NN, NC, DH).transpose(1, 0, 2)
    layers = _lightgcn_sc(x, src, dst, val)
    total = _sum4_tc(x, layers)
    return total.transpose(1, 0, 2).reshape(NN, D)
